# baseline (speedup 1.0000x reference)
import functools

import jax
import jax.numpy as jnp
from jax import lax
from jax.experimental import pallas as pl
from jax.experimental.pallas import tpu as pltpu
from jax.experimental.pallas import tpu_sc as plsc

MAXLEN = 200
EMBED = 64
BATCH = 1024

NC = 2
NS = 16
NW = NC * NS
LANES = 16

BTILE = 128
NBT = BATCH // BTILE
NUNITS = MAXLEN * NBT
PER_WORKER = NUNITS // NW
NPAIR = PER_WORKER // 2


NBUF = 6
NGRP = 48 // NBUF


def _body(idx_hbm, table_hbm, pos_hbm, out_hbm,
          idx_v, gath0, gath1, gath2, gath3, gath4, gath5, ob0, ob1, pos_v,
          g0, g1, g2, g3, g4, g5, o0, o1, psem):
    wid = lax.axis_index("s") * NC + lax.axis_index("c")
    ubase = wid * PER_WORKER

    gath = (gath0, gath1, gath2, gath3, gath4, gath5)
    obuf = (ob0, ob1)
    gsem = (g0, g1, g2, g3, g4, g5)
    osem = (o0, o1)

    pltpu.sync_copy(idx_hbm.at[pl.ds(ubase * BTILE, PER_WORKER * BTILE)],
                    idx_v)
    pos_cp = pltpu.async_copy(pos_hbm, pos_v, psem)

    iota = lax.iota(jnp.int32, LANES)
    dhv = [(iota + c * LANES) // 8 for c in range(EMBED // LANES)]
    dlv = [(iota + c * LANES) % 8 for c in range(EMBED // LANES)]

    def fetch(u, b):
        i = u - ubase
        pltpu.async_copy(table_hbm.at[idx_v.at[pl.ds(i * BTILE, BTILE)]],
                         gath[b], gsem[b])

    def wait_gather(b):
        pltpu.make_async_copy(table_hbm.at[idx_v.at[pl.ds(0, BTILE)]],
                              gath[b], gsem[b]).wait()

    def flush(u, b):
        t = u // NBT
        bt = u % NBT
        pltpu.async_copy(obuf[b].at[:, :, pl.ds(0, BTILE)],
                         out_hbm.at[t, :, bt], osem[b])

    def wait_out(b):
        pltpu.make_async_copy(obuf[b].at[:, :, pl.ds(0, BTILE)],
                              out_hbm.at[0, :, 0], osem[b]).wait()

    def transpose_add(u, gi, oi):
        t = u // NBT
        gb = gath[gi]
        ob = obuf[oi]
        prow = [pos_v[t, pl.ds(c * LANES, LANES)]
                for c in range(EMBED // LANES)]

        @plsc.parallel_loop(0, BTILE, unroll=4)
        def b_body(r):
            rsp = jnp.full((LANES,), r, jnp.int32)
            for c in range(EMBED // LANES):
                vals = gb[r, pl.ds(c * LANES, LANES)] + prow[c]
                plsc.store_scatter(ob, [dhv[c], dlv[c], rsp], vals)

    for j in range(NBUF):
        fetch(ubase + j, j)
    pos_cp.wait()

    def grp_body(g, carry):
        for j in range(NBUF):
            u = ubase + g * NBUF + j
            ob = j % 2
            wait_gather(j)

            @pl.when(g > 0)
            def _():
                wait_out(ob)

            transpose_add(u, j, ob)
            flush(u, ob)

            @pl.when(u + NBUF < ubase + PER_WORKER)
            def _():
                fetch(u + NBUF, j)
        return carry

    lax.fori_loop(0, NGRP, grp_body, 0)

    for j in range(PER_WORKER - NBUF * NGRP):
        u = ubase + NBUF * NGRP + j
        wait_gather(j)
        wait_out(j % 2)
        transpose_add(u, j, j % 2)
        flush(u, j % 2)
    for b in range(2):
        wait_out(b)
        wait_out(b)


@jax.jit
def kernel(inputs, token_emb, pos_emb):
    idx_t = jnp.swapaxes(inputs, 0, 1).astype(jnp.int32).reshape(-1)
    mesh = plsc.VectorSubcoreMesh(core_axis_name="c", subcore_axis_name="s")
    run = functools.partial(
        pl.kernel,
        out_type=jax.ShapeDtypeStruct((MAXLEN, 8, NBT, 8, BTILE), jnp.float32),
        mesh=mesh,
        scratch_types=[
            pltpu.VMEM((PER_WORKER * BTILE,), jnp.int32),
            pltpu.VMEM((BTILE, EMBED), jnp.float32),
            pltpu.VMEM((BTILE, EMBED), jnp.float32),
            pltpu.VMEM((BTILE, EMBED), jnp.float32),
            pltpu.VMEM((BTILE, EMBED), jnp.float32),
            pltpu.VMEM((BTILE, EMBED), jnp.float32),
            pltpu.VMEM((BTILE, EMBED), jnp.float32),
            pltpu.VMEM((8, 8, BTILE + 1), jnp.float32),
            pltpu.VMEM((8, 8, BTILE + 1), jnp.float32),
            pltpu.VMEM((MAXLEN, EMBED), jnp.float32),
            pltpu.SemaphoreType.DMA,
            pltpu.SemaphoreType.DMA,
            pltpu.SemaphoreType.DMA,
            pltpu.SemaphoreType.DMA,
            pltpu.SemaphoreType.DMA,
            pltpu.SemaphoreType.DMA,
            pltpu.SemaphoreType.DMA,
            pltpu.SemaphoreType.DMA,
            pltpu.SemaphoreType.DMA,
        ],
        compiler_params=pltpu.CompilerParams(
            use_tc_tiling_on_sc=False, needs_layout_passes=False),
    )(_body)
    out5 = run(idx_t, token_emb, pos_emb)
    return jnp.transpose(out5, (2, 4, 0, 1, 3)).reshape(BATCH, MAXLEN, EMBED)


# speedup vs baseline: 1.0010x; 1.0010x over previous
import functools

import jax
import jax.numpy as jnp
from jax import lax
from jax.experimental import pallas as pl
from jax.experimental.pallas import tpu as pltpu
from jax.experimental.pallas import tpu_sc as plsc

MAXLEN = 200
EMBED = 64
BATCH = 1024

NC = 2
NS = 16
NW = NC * NS
LANES = 16

BTILE = 128
NBT = BATCH // BTILE
NUNITS = MAXLEN * NBT
PER_WORKER = NUNITS // NW
NPAIR = PER_WORKER // 2


NBUF = 4
NGRP = 48 // NBUF


def _body(idx_hbm, table_hbm, pos_hbm, out_hbm,
          idx_v, gath0, gath1, gath2, gath3, ob0, ob1, pos_v,
          g0, g1, g2, g3, o0, o1, psem):
    wid = lax.axis_index("s") * NC + lax.axis_index("c")
    ubase = wid * PER_WORKER

    gath = (gath0, gath1, gath2, gath3)
    obuf = (ob0, ob1)
    gsem = (g0, g1, g2, g3)
    osem = (o0, o1)

    pltpu.sync_copy(idx_hbm.at[pl.ds(ubase * BTILE, PER_WORKER * BTILE)],
                    idx_v)
    pos_cp = pltpu.async_copy(pos_hbm, pos_v, psem)

    iota = lax.iota(jnp.int32, LANES)
    dhv = [(iota + c * LANES) // 8 for c in range(EMBED // LANES)]
    dlv = [(iota + c * LANES) % 8 for c in range(EMBED // LANES)]

    def fetch(u, b):
        i = u - ubase
        pltpu.async_copy(table_hbm.at[idx_v.at[pl.ds(i * BTILE, BTILE)]],
                         gath[b], gsem[b])

    def wait_gather(b):
        pltpu.make_async_copy(table_hbm.at[idx_v.at[pl.ds(0, BTILE)]],
                              gath[b], gsem[b]).wait()

    def flush(u, b):
        t = u // NBT
        bt = u % NBT
        pltpu.async_copy(obuf[b].at[:, :, pl.ds(0, BTILE)],
                         out_hbm.at[t, :, bt], osem[b])

    def wait_out(b):
        pltpu.make_async_copy(obuf[b].at[:, :, pl.ds(0, BTILE)],
                              out_hbm.at[0, :, 0], osem[b]).wait()

    def transpose_add(u, gi, oi):
        t = u // NBT
        gb = gath[gi]
        ob = obuf[oi]
        prow = [pos_v[t, pl.ds(c * LANES, LANES)]
                for c in range(EMBED // LANES)]

        @plsc.parallel_loop(0, BTILE, unroll=4)
        def b_body(r):
            rsp = jnp.full((LANES,), r, jnp.int32)
            for c in range(EMBED // LANES):
                vals = gb[r, pl.ds(c * LANES, LANES)] + prow[c]
                plsc.store_scatter(ob, [dhv[c], dlv[c], rsp], vals)

    for j in range(NBUF):
        fetch(ubase + j, j)
    pos_cp.wait()

    def grp_body(g, carry):
        for j in range(NBUF):
            u = ubase + g * NBUF + j
            ob = j % 2
            wait_gather(j)

            if j < 2:
                @pl.when(g > 0)
                def _():
                    wait_out(ob)
            else:
                wait_out(ob)

            transpose_add(u, j, ob)
            flush(u, ob)

            @pl.when(u + NBUF < ubase + PER_WORKER)
            def _():
                fetch(u + NBUF, j)
        return carry

    lax.fori_loop(0, NGRP, grp_body, 0)

    for j in range(PER_WORKER - NBUF * NGRP):
        u = ubase + NBUF * NGRP + j
        wait_gather(j)
        wait_out(j % 2)
        transpose_add(u, j, j % 2)
        flush(u, j % 2)
    for b in range(2):
        wait_out(b)


@jax.jit
def kernel(inputs, token_emb, pos_emb):
    idx_t = jnp.swapaxes(inputs, 0, 1).astype(jnp.int32).reshape(-1)
    mesh = plsc.VectorSubcoreMesh(core_axis_name="c", subcore_axis_name="s")
    run = functools.partial(
        pl.kernel,
        out_type=jax.ShapeDtypeStruct((MAXLEN, 8, NBT, 8, BTILE), jnp.float32),
        mesh=mesh,
        scratch_types=[
            pltpu.VMEM((PER_WORKER * BTILE,), jnp.int32),
            pltpu.VMEM((BTILE, EMBED), jnp.float32),
            pltpu.VMEM((BTILE, EMBED), jnp.float32),
            pltpu.VMEM((BTILE, EMBED), jnp.float32),
            pltpu.VMEM((BTILE, EMBED), jnp.float32),
            pltpu.VMEM((8, 8, BTILE + 1), jnp.float32),
            pltpu.VMEM((8, 8, BTILE + 1), jnp.float32),
            pltpu.VMEM((MAXLEN, EMBED), jnp.float32),
            pltpu.SemaphoreType.DMA,
            pltpu.SemaphoreType.DMA,
            pltpu.SemaphoreType.DMA,
            pltpu.SemaphoreType.DMA,
            pltpu.SemaphoreType.DMA,
            pltpu.SemaphoreType.DMA,
            pltpu.SemaphoreType.DMA,
        ],
        compiler_params=pltpu.CompilerParams(
            use_tc_tiling_on_sc=False, needs_layout_passes=False),
    )(_body)
    out5 = run(idx_t, token_emb, pos_emb)
    return jnp.transpose(out5, (2, 4, 0, 1, 3)).reshape(BATCH, MAXLEN, EMBED)


# speedup vs baseline: 1.0052x; 1.0042x over previous
import functools

import jax
import jax.numpy as jnp
from jax import lax
from jax.experimental import pallas as pl
from jax.experimental.pallas import tpu as pltpu
from jax.experimental.pallas import tpu_sc as plsc

MAXLEN = 200
EMBED = 64
BATCH = 1024

NC = 2
NS = 16
NW = NC * NS
LANES = 16

BTILE = 128
NBT = BATCH // BTILE
NUNITS = MAXLEN * NBT
PER_WORKER = NUNITS // NW

NBUF = 4
NGRP = 48 // NBUF


def _body(idx_hbm, table_hbm, pos_hbm, out_hbm,
          idx_v, gath0, gath1, gath2, gath3, ob0, ob1, pos_v,
          g0, g1, g2, g3, o0, o1, psem):
    wid = lax.axis_index("s") * NC + lax.axis_index("c")
    ubase = wid * PER_WORKER

    gath = (gath0, gath1, gath2, gath3)
    obuf = (ob0, ob1)
    gsem = (g0, g1, g2, g3)
    osem = (o0, o1)

    pltpu.sync_copy(idx_hbm.at[pl.ds(ubase * BTILE, PER_WORKER * BTILE)],
                    idx_v)
    pos_cp = pltpu.async_copy(pos_hbm, pos_v, psem)

    iota = lax.iota(jnp.int32, LANES)
    dhv = [(iota + c * LANES) // 8 for c in range(EMBED // LANES)]
    dlv = [(iota + c * LANES) % 8 for c in range(EMBED // LANES)]

    def fetch(u, b):
        i = u - ubase
        pltpu.async_copy(table_hbm.at[idx_v.at[pl.ds(i * BTILE, BTILE)]],
                         gath[b], gsem[b])

    def wait_gather(b):
        pltpu.make_async_copy(table_hbm.at[idx_v.at[pl.ds(0, BTILE)]],
                              gath[b], gsem[b]).wait()

    def flush(u, b):
        t = u // NBT
        bt = u % NBT
        pltpu.async_copy(obuf[b].at[:, :, pl.ds(0, BTILE)],
                         out_hbm.at[t, :, bt], osem[b])

    def wait_out(b):
        pltpu.make_async_copy(obuf[b].at[:, :, pl.ds(0, BTILE)],
                              out_hbm.at[0, :, 0], osem[b]).wait()

    def transpose_add(u, gi, oi):
        t = u // NBT
        gb = gath[gi]
        ob = obuf[oi]
        prow = [pos_v[t, pl.ds(c * LANES, LANES)]
                for c in range(EMBED // LANES)]

        @plsc.parallel_loop(0, BTILE, unroll=4)
        def b_body(r):
            rsp = jnp.full((LANES,), r, jnp.int32)
            for c in range(EMBED // LANES):
                vals = gb[r, pl.ds(c * LANES, LANES)] + prow[c]
                plsc.store_scatter(ob, [dhv[c], dlv[c], rsp], vals)

    for j in range(NBUF):
        fetch(ubase + j, j)
    pos_cp.wait()

    def grp_body(g, carry):
        for j in range(NBUF):
            u = ubase + g * NBUF + j
            ob = j % 2
            wait_gather(j)

            if j < 2:
                @pl.when(g > 0)
                def _():
                    wait_out(ob)
            else:
                wait_out(ob)

            transpose_add(u, j, ob)
            flush(u, ob)

            @pl.when(u + NBUF < ubase + PER_WORKER)
            def _():
                fetch(u + NBUF, j)
        return carry

    lax.fori_loop(0, NGRP, grp_body, 0)

    for j in range(PER_WORKER - NBUF * NGRP):
        u = ubase + NBUF * NGRP + j
        wait_gather(j)
        wait_out(j % 2)
        transpose_add(u, j, j % 2)
        flush(u, j % 2)
    for b in range(2):
        wait_out(b)


@jax.jit
def kernel(inputs, token_emb, pos_emb):
    idx_t = jnp.swapaxes(inputs, 0, 1).astype(jnp.int32).reshape(-1)
    mesh = plsc.VectorSubcoreMesh(core_axis_name="c", subcore_axis_name="s")
    run = functools.partial(
        pl.kernel,
        out_type=jax.ShapeDtypeStruct((MAXLEN, 8, NBT, 8, BTILE), jnp.float32),
        mesh=mesh,
        scratch_types=[
            pltpu.VMEM((PER_WORKER * BTILE,), jnp.int32),
            pltpu.VMEM((BTILE, EMBED), jnp.float32),
            pltpu.VMEM((BTILE, EMBED), jnp.float32),
            pltpu.VMEM((BTILE, EMBED), jnp.float32),
            pltpu.VMEM((BTILE, EMBED), jnp.float32),
            pltpu.VMEM((8, 8, BTILE + 1), jnp.float32),
            pltpu.VMEM((8, 8, BTILE + 1), jnp.float32),
            pltpu.VMEM((MAXLEN, EMBED), jnp.float32),
            pltpu.SemaphoreType.DMA,
            pltpu.SemaphoreType.DMA,
            pltpu.SemaphoreType.DMA,
            pltpu.SemaphoreType.DMA,
            pltpu.SemaphoreType.DMA,
            pltpu.SemaphoreType.DMA,
            pltpu.SemaphoreType.DMA,
        ],
        compiler_params=pltpu.CompilerParams(
            use_tc_tiling_on_sc=False, needs_layout_passes=False),
    )(_body)
    out5 = run(idx_t, token_emb, pos_emb)
    return jnp.transpose(out5, (2, 4, 0, 1, 3)).reshape(BATCH, MAXLEN, EMBED)
